# baseline (speedup 1.0000x reference)
import jax
import jax.numpy as jnp
from jax import lax
from jax.experimental import pallas as pl
from jax.experimental.pallas import tpu as pltpu, tpu_sc as plsc

B, S, H, V, P, T = 4, 2048, 768, 100000, 4096, 2
LN_EPS = 1e-12

NC, NS, L = 2, 16, 16
NW = NC * NS
BS = B * S
TPW = BS // NW
CH = 32
NCHUNK = TPW // CH
HV = H // L


def _lane_shuffle(v, perm):
    return lax.gather(
        v, perm[:, None],
        dimension_numbers=lax.GatherDimensionNumbers(
            offset_dims=(), collapsed_slice_dims=(0,), start_index_map=(0,)),
        slice_sizes=(1,),
        mode=lax.GatherScatterMode.PROMISE_IN_BOUNDS)


def _body(ids_hbm, word_hbm, tt_hbm, pos_hbm, gamma_hbm, beta_hbm, out_hbm,
          idx_v, rows_v, pos_v, tt_v, gamma_v, beta_v, sem):
    wid = lax.axis_index("s") * NC + lax.axis_index("c")
    base = wid * TPW
    p0 = lax.rem(base, S)

    pltpu.sync_copy(tt_hbm.at[0], tt_v)
    pltpu.sync_copy(gamma_hbm, gamma_v)
    pltpu.sync_copy(beta_hbm, beta_v)

    def chunk_body(c, _):
        tok0 = base + c * CH
        pltpu.sync_copy(ids_hbm.at[pl.ds(tok0, CH)], idx_v)
        gather = pltpu.async_copy(word_hbm.at[idx_v], rows_v, sem)
        pltpu.sync_copy(pos_hbm.at[pl.ds(p0 + c * CH, CH)], pos_v)
        gather.wait()

        def token_body(t, _):
            zeros = jnp.zeros((L,), jnp.float32)

            def h1(h, carry):
                s_v, q_v = carry
                off = pl.ds(h * L, L)
                v = rows_v[t, off] + pos_v[t, off] + tt_v[off]
                rows_v[t, off] = v
                return (s_v + v, q_v + v * v)

            s_v, q_v = lax.fori_loop(0, HV, h1, (zeros, zeros))
            iota = lax.iota(jnp.int32, L)
            for sh in (8, 4, 2, 1):
                perm = lax.bitwise_xor(iota, sh)
                s_v = s_v + _lane_shuffle(s_v, perm)
                q_v = q_v + _lane_shuffle(q_v, perm)
            mean_b = s_v * (1.0 / H)
            var_b = q_v * (1.0 / H) - mean_b * mean_b + LN_EPS
            i = lax.bitcast_convert_type(var_b, jnp.int32)
            i = 0x5F3759DF - lax.shift_right_logical(i, 1)
            y = lax.bitcast_convert_type(i, jnp.float32)
            for _ in range(3):
                y = y * (1.5 - 0.5 * var_b * y * y)

            def h2(h, carry):
                off = pl.ds(h * L, L)
                v = (rows_v[t, off] - mean_b) * y
                rows_v[t, off] = v * gamma_v[off] + beta_v[off]
                return carry

            lax.fori_loop(0, HV, h2, 0)
            return 0

        lax.fori_loop(0, CH, token_body, 0)
        pltpu.sync_copy(rows_v, out_hbm.at[pl.ds(tok0, CH)])
        return 0

    lax.fori_loop(0, NCHUNK, chunk_body, 0)


@jax.jit
def _run(ids_flat, word_emb, token_type_emb, pos_emb, gamma, beta):
    mesh = plsc.VectorSubcoreMesh(core_axis_name="c", subcore_axis_name="s")
    kfn = pl.kernel(
        _body,
        out_type=jax.ShapeDtypeStruct((BS, H), jnp.float32),
        mesh=mesh,
        scratch_types=[
            pltpu.VMEM((CH,), jnp.int32),
            pltpu.VMEM((CH, H), jnp.float32),
            pltpu.VMEM((CH, H), jnp.float32),
            pltpu.VMEM((H,), jnp.float32),
            pltpu.VMEM((H,), jnp.float32),
            pltpu.VMEM((H,), jnp.float32),
            pltpu.SemaphoreType.DMA,
        ],
    )
    return kfn(ids_flat, word_emb, token_type_emb, pos_emb, gamma, beta)


def kernel(input_ids, word_emb, token_type_emb, pos_emb, gamma, beta):
    ids_flat = input_ids.reshape(BS).astype(jnp.int32)
    out = _run(ids_flat, word_emb, token_type_emb, pos_emb, gamma, beta)
    return out.reshape(B, S, H)
